# baseline (speedup 1.0000x reference)
import jax
import jax.numpy as jnp
from jax.experimental import pallas as pl
from jax.experimental.pallas import tpu as pltpu

_VMEM = 48 * 1024 * 1024
_K = 5


def _band_mats(w, W, OW2):
    Cout, Cin = w.shape[0], w.shape[1]
    wi = jnp.arange(W)[:, None, None]
    ow = jnp.arange(OW2)[None, None, :]
    p = jnp.arange(2)[None, :, None]
    taps = jnp.stack([(wi == 2 * ow + p + kw).astype(w.dtype)
                      for kw in range(_K)])
    a = jnp.einsum("kwpv,oihk->hiwpov", taps, w)
    return a.reshape(_K, Cin * W, 2 * Cout * OW2)


def _conv_block(rows, a_ref, p_ref, r_ref, oh):
    acc = jnp.dot(rows(0), a_ref[0], preferred_element_type=jnp.float32)
    for kh in range(1, _K):
        acc += jnp.dot(rows(kh), a_ref[kh],
                       preferred_element_type=jnp.float32)
    n2 = acc.shape[1] // 2
    t = jnp.maximum(acc[:, 0:n2], acc[:, n2:])
    u = jnp.dot(p_ref[...], t, preferred_element_type=jnp.float32)
    half = u[0:oh // 2] + jnp.abs(u[oh // 2:oh])
    return jnp.maximum(half + r_ref[...], 0.0)


def _tower_kernel(x_ref, a1_ref, p1_ref, r1_ref, a2_ref, p2_ref, r2_ref,
                  a3_ref, p3_ref, r3_ref, o_ref):
    B = x_ref.shape[0]
    for b in range(B):
        z1 = _conv_block(lambda kh: x_ref[b, kh:kh + 192, :],
                         a1_ref, p1_ref, r1_ref, 192)
        z2 = _conv_block(lambda kh: z1[kh:kh + 92],
                         a2_ref, p2_ref, r2_ref, 92)
        o_ref[b] = _conv_block(lambda kh: z2[kh:kh + 42],
                               a3_ref, p3_ref, r3_ref, 42)


def _mlp_kernel(f_ref, w1_ref, b1_ref, w2_ref, b2_ref, w3_ref, b3_ref,
                o_ref):
    h = jnp.dot(f_ref[...], w1_ref[...],
                preferred_element_type=jnp.float32) + b1_ref[...]
    h = jnp.maximum(h, 0.0)
    h = jnp.dot(h, w2_ref[...],
                preferred_element_type=jnp.float32) + b2_ref[...]
    h = jnp.maximum(h, 0.0)
    o_ref[...] = jnp.dot(h, w3_ref[...],
                         preferred_element_type=jnp.float32) + b3_ref[...]


def kernel(x, conv1_w, conv1_b, conv2_w, conv2_b, conv3_w, conv3_b,
           fc1_w, fc1_b, fc2_w, fc2_b, fc3_w, fc3_b):
    N = x.shape[0]
    x2 = x.reshape(N, 196, 196)

    a1 = _band_mats(conv1_w, 196, 96)
    a2 = _band_mats(conv2_w, 96, 46)
    a3 = _band_mats(conv3_w, 46, 21)
    r1 = jnp.repeat(conv1_b, 96).reshape(1, 6 * 96)
    r2 = jnp.repeat(conv2_b, 46).reshape(1, 6 * 46)
    r3 = jnp.repeat(conv3_b, 21).reshape(1, 16 * 21)

    def pool_mat(oh):
        i2 = 2 * jnp.arange(oh // 2)[:, None]
        c = jnp.arange(oh)[None, :]
        ps = 0.5 * ((c == i2) + (c == i2 + 1)).astype(jnp.float32)
        pd = 0.5 * ((c == i2).astype(jnp.float32)
                    - (c == i2 + 1).astype(jnp.float32))
        return jnp.concatenate([ps, pd], axis=0)

    p1, p2, p3 = pool_mat(192), pool_mat(92), pool_mat(42)

    B = next(bb for bb in (8, 4, 2, 1) if N % bb == 0)
    full = lambda arr: pl.BlockSpec(arr.shape, lambda i: (0,) * arr.ndim)
    f32 = jnp.float32
    feats = pl.pallas_call(
        _tower_kernel,
        out_shape=jax.ShapeDtypeStruct((N, 21, 16 * 21), f32),
        grid=(N // B,),
        in_specs=[pl.BlockSpec((B, 196, 196), lambda i: (i, 0, 0)),
                  full(a1), full(p1), full(r1), full(a2), full(p2), full(r2),
                  full(a3), full(p3), full(r3)],
        out_specs=pl.BlockSpec((B, 21, 16 * 21), lambda i: (i, 0, 0)),
        compiler_params=pltpu.CompilerParams(
            dimension_semantics=("parallel",), vmem_limit_bytes=_VMEM),
    )(x2, a1, p1, r1, a2, p2, r2, a3, p3, r3)

    f = feats.reshape(N, 21 * 16 * 21)
    w1p = (fc1_w.reshape(120, 16, 21, 21).transpose(0, 2, 1, 3)
           .reshape(120, 7056).T)

    MB = N // 2 if N % 2 == 0 else N
    out = pl.pallas_call(
        _mlp_kernel,
        out_shape=jax.ShapeDtypeStruct((N, 5), f32),
        grid=(N // MB,),
        in_specs=[pl.BlockSpec((MB, 7056), lambda i: (i, 0)),
                  full(w1p), full(fc1_b.reshape(1, 120)),
                  full(fc2_w.T), full(fc2_b.reshape(1, 32)),
                  full(fc3_w.T), full(fc3_b.reshape(1, 5))],
        out_specs=pl.BlockSpec((MB, 5), lambda i: (i, 0)),
        compiler_params=pltpu.CompilerParams(
            dimension_semantics=("parallel",), vmem_limit_bytes=_VMEM),
    )(f, w1p, fc1_b.reshape(1, 120), fc2_w.T, fc2_b.reshape(1, 32),
      fc3_w.T, fc3_b.reshape(1, 5))
    return out


# speedup vs baseline: 1.0345x; 1.0345x over previous
import jax
import jax.numpy as jnp
from jax.experimental import pallas as pl
from jax.experimental.pallas import tpu as pltpu

_VMEM = 48 * 1024 * 1024
_K = 5


def _band_mats(w, W, OW2):
    Cout, Cin = w.shape[0], w.shape[1]
    wi = jnp.arange(W)[:, None, None]
    ow = jnp.arange(OW2)[None, None, :]
    p = jnp.arange(2)[None, :, None]
    taps = jnp.stack([(wi == 2 * ow + p + kw).astype(w.dtype)
                      for kw in range(_K)])
    a = jnp.einsum("kwpv,oihk->hiwpov", taps, w)
    return a.reshape(_K, Cin * W, 2 * Cout * OW2)


def _conv_block(rows, a_ref, p_ref, r_ref, oh):
    acc = jnp.dot(rows(0), a_ref[0], preferred_element_type=jnp.float32)
    for kh in range(1, _K):
        acc += jnp.dot(rows(kh), a_ref[kh],
                       preferred_element_type=jnp.float32)
    n2 = acc.shape[1] // 2
    t = jnp.maximum(acc[:, 0:n2], acc[:, n2:])
    u = jnp.dot(p_ref[...], t, preferred_element_type=jnp.float32)
    half = u[0:oh // 2] + jnp.abs(u[oh // 2:oh])
    return jnp.maximum(half + r_ref[...], 0.0)


def _tower_kernel(x_ref, a1_ref, p1_ref, r1_ref, a2_ref, p2_ref, r2_ref,
                  a3_ref, p3_ref, r3_ref, o_ref):
    B = x_ref.shape[0]
    for b in range(B):
        z1 = _conv_block(lambda kh: x_ref[b, kh:kh + 192, :],
                         a1_ref, p1_ref, r1_ref, 192)
        z2 = _conv_block(lambda kh: z1[kh:kh + 92],
                         a2_ref, p2_ref, r2_ref, 92)
        o_ref[b] = _conv_block(lambda kh: z2[kh:kh + 42],
                               a3_ref, p3_ref, r3_ref, 42)


def _mlp_kernel(f_ref, w1_ref, b1_ref, w2_ref, b2_ref, w3_ref, b3_ref,
                o_ref):
    h = jnp.dot(f_ref[...], w1_ref[...],
                preferred_element_type=jnp.float32) + b1_ref[...]
    h = jnp.maximum(h, 0.0)
    h = jnp.dot(h, w2_ref[...],
                preferred_element_type=jnp.float32) + b2_ref[...]
    h = jnp.maximum(h, 0.0)
    o_ref[...] = jnp.dot(h, w3_ref[...],
                         preferred_element_type=jnp.float32) + b3_ref[...]


def kernel(x, conv1_w, conv1_b, conv2_w, conv2_b, conv3_w, conv3_b,
           fc1_w, fc1_b, fc2_w, fc2_b, fc3_w, fc3_b):
    N = x.shape[0]
    x2 = x.reshape(N, 196, 196)

    a1 = _band_mats(conv1_w, 196, 96)
    a2 = _band_mats(conv2_w, 96, 46)
    a3 = _band_mats(conv3_w, 46, 21)
    r1 = jnp.repeat(conv1_b, 96).reshape(1, 6 * 96)
    r2 = jnp.repeat(conv2_b, 46).reshape(1, 6 * 46)
    r3 = jnp.repeat(conv3_b, 21).reshape(1, 16 * 21)

    def pool_mat(oh):
        i2 = 2 * jnp.arange(oh // 2)[:, None]
        c = jnp.arange(oh)[None, :]
        ps = 0.5 * ((c == i2) + (c == i2 + 1)).astype(jnp.float32)
        pd = 0.5 * ((c == i2).astype(jnp.float32)
                    - (c == i2 + 1).astype(jnp.float32))
        return jnp.concatenate([ps, pd], axis=0)

    p1, p2, p3 = pool_mat(192), pool_mat(92), pool_mat(42)

    B = next(bb for bb in (4, 2, 1) if N % bb == 0)
    full = lambda arr: pl.BlockSpec(arr.shape, lambda i: (0,) * arr.ndim)
    f32 = jnp.float32
    feats = pl.pallas_call(
        _tower_kernel,
        out_shape=jax.ShapeDtypeStruct((N, 21, 16 * 21), f32),
        grid=(N // B,),
        in_specs=[pl.BlockSpec((B, 196, 196), lambda i: (i, 0, 0)),
                  full(a1), full(p1), full(r1), full(a2), full(p2), full(r2),
                  full(a3), full(p3), full(r3)],
        out_specs=pl.BlockSpec((B, 21, 16 * 21), lambda i: (i, 0, 0)),
        compiler_params=pltpu.CompilerParams(
            dimension_semantics=("parallel",), vmem_limit_bytes=_VMEM),
    )(x2, a1, p1, r1, a2, p2, r2, a3, p3, r3)

    f = feats.reshape(N, 21 * 16 * 21)
    w1p = (fc1_w.reshape(120, 16, 21, 21).transpose(0, 2, 1, 3)
           .reshape(120, 7056).T)

    MB = N // 2 if N % 2 == 0 else N
    out = pl.pallas_call(
        _mlp_kernel,
        out_shape=jax.ShapeDtypeStruct((N, 5), f32),
        grid=(N // MB,),
        in_specs=[pl.BlockSpec((MB, 7056), lambda i: (i, 0)),
                  full(w1p), full(fc1_b.reshape(1, 120)),
                  full(fc2_w.T), full(fc2_b.reshape(1, 32)),
                  full(fc3_w.T), full(fc3_b.reshape(1, 5))],
        out_specs=pl.BlockSpec((MB, 5), lambda i: (i, 0)),
        compiler_params=pltpu.CompilerParams(
            dimension_semantics=("parallel",), vmem_limit_bytes=_VMEM),
    )(f, w1p, fc1_b.reshape(1, 120), fc2_w.T, fc2_b.reshape(1, 32),
      fc3_w.T, fc3_b.reshape(1, 5))
    return out


# speedup vs baseline: 1.0634x; 1.0279x over previous
import jax
import jax.numpy as jnp
from jax.experimental import pallas as pl
from jax.experimental.pallas import tpu as pltpu

_VMEM = 48 * 1024 * 1024
_K = 5


def _band_mats(w, W, OW2):
    Cout, Cin = w.shape[0], w.shape[1]
    wi = jnp.arange(W)[:, None, None]
    ow = jnp.arange(OW2)[None, None, :]
    p = jnp.arange(2)[None, :, None]
    taps = jnp.stack([(wi == 2 * ow + p + kw).astype(w.dtype)
                      for kw in range(_K)])
    a = jnp.einsum("kwpv,oihk->hiwpov", taps, w)
    return a.reshape(_K, Cin * W, 2 * Cout * OW2)


def _conv_block(rows, a_ref, p_ref, r_ref, oh):
    acc = jnp.dot(rows(0).astype(a_ref.dtype), a_ref[0],
                  preferred_element_type=jnp.float32)
    for kh in range(1, _K):
        acc += jnp.dot(rows(kh).astype(a_ref.dtype), a_ref[kh],
                       preferred_element_type=jnp.float32)
    n2 = acc.shape[1] // 2
    t = jnp.maximum(acc[:, 0:n2], acc[:, n2:])
    u = jnp.dot(p_ref[...], t, preferred_element_type=jnp.float32)
    half = u[0:oh // 2] + jnp.abs(u[oh // 2:oh])
    return jnp.maximum(half + r_ref[...], 0.0)


def _tower_kernel(x_ref, a1_ref, p1_ref, r1_ref, a2_ref, p2_ref, r2_ref,
                  a3_ref, p3_ref, r3_ref, o_ref):
    B = x_ref.shape[0]
    for b in range(B):
        z1 = _conv_block(lambda kh: x_ref[b, kh:kh + 192, :],
                         a1_ref, p1_ref, r1_ref, 192)
        z2 = _conv_block(lambda kh: z1[kh:kh + 92],
                         a2_ref, p2_ref, r2_ref, 92)
        o_ref[b] = _conv_block(lambda kh: z2[kh:kh + 42],
                               a3_ref, p3_ref, r3_ref, 42)


def _mlp_kernel(f_ref, w1_ref, b1_ref, w2_ref, b2_ref, w3_ref, b3_ref,
                o_ref):
    h = jnp.dot(f_ref[...], w1_ref[...],
                preferred_element_type=jnp.float32) + b1_ref[...]
    h = jnp.maximum(h, 0.0)
    h = jnp.dot(h, w2_ref[...],
                preferred_element_type=jnp.float32) + b2_ref[...]
    h = jnp.maximum(h, 0.0)
    o_ref[...] = jnp.dot(h, w3_ref[...],
                         preferred_element_type=jnp.float32) + b3_ref[...]


def kernel(x, conv1_w, conv1_b, conv2_w, conv2_b, conv3_w, conv3_b,
           fc1_w, fc1_b, fc2_w, fc2_b, fc3_w, fc3_b):
    N = x.shape[0]
    x2 = x.reshape(N, 196, 196)

    a1 = _band_mats(conv1_w, 196, 96).astype(jnp.bfloat16)
    a2 = _band_mats(conv2_w, 96, 46).astype(jnp.bfloat16)
    a3 = _band_mats(conv3_w, 46, 21).astype(jnp.bfloat16)
    r1 = jnp.repeat(conv1_b, 96).reshape(1, 6 * 96)
    r2 = jnp.repeat(conv2_b, 46).reshape(1, 6 * 46)
    r3 = jnp.repeat(conv3_b, 21).reshape(1, 16 * 21)

    def pool_mat(oh):
        i2 = 2 * jnp.arange(oh // 2)[:, None]
        c = jnp.arange(oh)[None, :]
        ps = 0.5 * ((c == i2) + (c == i2 + 1)).astype(jnp.float32)
        pd = 0.5 * ((c == i2).astype(jnp.float32)
                    - (c == i2 + 1).astype(jnp.float32))
        return jnp.concatenate([ps, pd], axis=0)

    p1, p2, p3 = pool_mat(192), pool_mat(92), pool_mat(42)

    B = next(bb for bb in (4, 2, 1) if N % bb == 0)
    full = lambda arr: pl.BlockSpec(arr.shape, lambda i: (0,) * arr.ndim)
    f32 = jnp.float32
    feats = pl.pallas_call(
        _tower_kernel,
        out_shape=jax.ShapeDtypeStruct((N, 21, 16 * 21), f32),
        grid=(N // B,),
        in_specs=[pl.BlockSpec((B, 196, 196), lambda i: (i, 0, 0)),
                  full(a1), full(p1), full(r1), full(a2), full(p2), full(r2),
                  full(a3), full(p3), full(r3)],
        out_specs=pl.BlockSpec((B, 21, 16 * 21), lambda i: (i, 0, 0)),
        compiler_params=pltpu.CompilerParams(
            dimension_semantics=("parallel",), vmem_limit_bytes=_VMEM),
    )(x2, a1, p1, r1, a2, p2, r2, a3, p3, r3)

    f = feats.reshape(N, 21 * 16 * 21)
    w1p = (fc1_w.reshape(120, 16, 21, 21).transpose(0, 2, 1, 3)
           .reshape(120, 7056).T)

    MB = N // 2 if N % 2 == 0 else N
    out = pl.pallas_call(
        _mlp_kernel,
        out_shape=jax.ShapeDtypeStruct((N, 5), f32),
        grid=(N // MB,),
        in_specs=[pl.BlockSpec((MB, 7056), lambda i: (i, 0)),
                  full(w1p), full(fc1_b.reshape(1, 120)),
                  full(fc2_w.T), full(fc2_b.reshape(1, 32)),
                  full(fc3_w.T), full(fc3_b.reshape(1, 5))],
        out_specs=pl.BlockSpec((MB, 5), lambda i: (i, 0)),
        compiler_params=pltpu.CompilerParams(
            dimension_semantics=("parallel",), vmem_limit_bytes=_VMEM),
    )(f, w1p, fc1_b.reshape(1, 120), fc2_w.T, fc2_b.reshape(1, 32),
      fc3_w.T, fc3_b.reshape(1, 5))
    return out
